# Initial kernel scaffold; baseline (speedup 1.0000x reference)
import jax
import jax.numpy as jnp
from jax.experimental import pallas as pl


def kernel(x, edge_idx, edge_attr, W, b, gamma, beta):
    raise NotImplementedError("write your pallas kernel here")


# speedup vs baseline: 3.0407x; 3.0407x over previous
import functools

import jax
import jax.numpy as jnp
from jax import lax
from jax.experimental import pallas as pl
from jax.experimental.pallas import tpu as pltpu
from jax.experimental.pallas import tpu_sc as plsc

ALPHA = 0.01
EPS = 1e-5

NC = 2
NS = 16
NW = NC * NS
LANES = 16
CHUNK = 128


def _cdiv(a, b):
  return (a + b - 1) // b


def _deg_kernel(n_nodes, e_pad):
  epw = e_pad // NW
  n_chunks = epw // CHUNK
  rows_per_tile = n_nodes // NS

  @functools.partial(
      pl.kernel,
      out_type=jax.ShapeDtypeStruct((NC, n_nodes, LANES), jnp.float32),
      mesh=plsc.VectorSubcoreMesh(core_axis_name="c", subcore_axis_name="s", num_cores=NC, num_subcores=NS),
      compiler_params=pltpu.CompilerParams(use_tc_tiling_on_sc=False, needs_layout_passes=False),
      scratch_types=[
          pltpu.MemorySpace.VMEM_SHARED((n_nodes, LANES), jnp.float32),
          pltpu.VMEM((rows_per_tile, LANES), jnp.float32),
          pltpu.VMEM((CHUNK, LANES), jnp.float32),
          pltpu.VMEM((CHUNK,), jnp.int32),
          pltpu.VMEM((CHUNK,), jnp.int32),
          pltpu.VMEM((CHUNK,), jnp.float32),
      ],
  )
  def deg(src_hbm, dst_hbm, attr_hbm, part_hbm,
          acc_sh, zero_v, rows_v, src_v, dst_v, attr_v):
    c = lax.axis_index("c")
    s = lax.axis_index("s")
    wid = s * NC + c
    base = wid * epw
    r0 = s * rows_per_tile

    @pl.loop(0, rows_per_tile)
    def _(r):
      zero_v[r, pl.ds(0, LANES)] = jnp.zeros((LANES,), jnp.float32)

    pltpu.sync_copy(zero_v, acc_sh.at[pl.ds(r0, rows_per_tile)])
    plsc.subcore_barrier()

    @pl.loop(0, n_chunks)
    def _(k):
      e0 = base + k * CHUNK
      pltpu.sync_copy(src_hbm.at[pl.ds(e0, CHUNK)], src_v)
      pltpu.sync_copy(dst_hbm.at[pl.ds(e0, CHUNK)], dst_v)
      pltpu.sync_copy(attr_hbm.at[pl.ds(e0, CHUNK)], attr_v)

      @pl.loop(0, CHUNK // LANES)
      def _(g):
        sl = pl.ds(g * LANES, LANES)
        ew = jnp.where(src_v[sl] == dst_v[sl], 0.0, attr_v[sl])
        for l in range(LANES):
          rows_v[g * LANES + l, pl.ds(0, LANES)] = jnp.full(
              (LANES,), ew[l], jnp.float32)

      pltpu.sync_copy(rows_v, acc_sh.at[src_v], add=True)

    plsc.subcore_barrier()
    pltpu.sync_copy(acc_sh.at[pl.ds(r0, rows_per_tile)],
                    part_hbm.at[c, pl.ds(r0, rows_per_tile)])

  return deg


def _norm_kernel(n_nodes, e_pad):
  epw = e_pad // NW
  n_chunks = epw // CHUNK

  @functools.partial(
      pl.kernel,
      out_type=jax.ShapeDtypeStruct((e_pad,), jnp.float32),
      mesh=plsc.VectorSubcoreMesh(core_axis_name="c", subcore_axis_name="s", num_cores=NC, num_subcores=NS),
      compiler_params=pltpu.CompilerParams(use_tc_tiling_on_sc=False, needs_layout_passes=False),
      scratch_types=[
          pltpu.VMEM((n_nodes,), jnp.float32),
          pltpu.VMEM((CHUNK,), jnp.int32),
          pltpu.VMEM((CHUNK,), jnp.int32),
          pltpu.VMEM((CHUNK,), jnp.float32),
          pltpu.VMEM((CHUNK,), jnp.float32),
      ],
  )
  def norm(src_hbm, dst_hbm, attr_hbm, dis_hbm, norm_hbm,
           dis_v, src_v, dst_v, attr_v, norm_v):
    c = lax.axis_index("c")
    s = lax.axis_index("s")
    wid = s * NC + c
    base = wid * epw
    pltpu.sync_copy(dis_hbm, dis_v)

    @pl.loop(0, n_chunks)
    def _(k):
      e0 = base + k * CHUNK
      pltpu.sync_copy(src_hbm.at[pl.ds(e0, CHUNK)], src_v)
      pltpu.sync_copy(dst_hbm.at[pl.ds(e0, CHUNK)], dst_v)
      pltpu.sync_copy(attr_hbm.at[pl.ds(e0, CHUNK)], attr_v)
      for j in range(CHUNK // LANES):
        sl = pl.ds(j * LANES, LANES)
        si = src_v[sl]
        di = dst_v[sl]
        av = attr_v[sl]
        dsrc = plsc.load_gather(dis_v, [si])
        ddst = plsc.load_gather(dis_v, [di])
        ew = jnp.where(si == di, 0.0, av)
        norm_v[sl] = -dsrc * ew * ddst
      pltpu.sync_copy(norm_v, norm_hbm.at[pl.ds(e0, CHUNK)])

  return norm


def _lap_kernel(n_nodes, e_pad, dh):
  epw = e_pad // NS
  n_chunks = epw // CHUNK
  rows_per_tile = n_nodes // NS
  zrows = 125

  @functools.partial(
      pl.kernel,
      out_type=jax.ShapeDtypeStruct((n_nodes, NC, dh), jnp.float32),
      mesh=plsc.VectorSubcoreMesh(core_axis_name="c", subcore_axis_name="s", num_cores=NC, num_subcores=NS),
      compiler_params=pltpu.CompilerParams(use_tc_tiling_on_sc=False, needs_layout_passes=False),
      scratch_types=[
          pltpu.MemorySpace.VMEM_SHARED((n_nodes, dh), jnp.float32),
          pltpu.VMEM((CHUNK, dh), jnp.float32),
          pltpu.VMEM((zrows, dh), jnp.float32),
          pltpu.VMEM((CHUNK,), jnp.int32),
          pltpu.VMEM((CHUNK,), jnp.int32),
          pltpu.VMEM((CHUNK,), jnp.float32),
      ],
  )
  def lap(v2_hbm, gidx_hbm, dst_hbm, norm_hbm, out_hbm,
          acc_sh, rows_v, zero_v, idx_v, dsti_v, norm_v):
    c = lax.axis_index("c")
    s = lax.axis_index("s")
    base = s * epw
    r0 = s * rows_per_tile

    @pl.loop(0, zrows)
    def _(r):
      for j in range(dh // LANES):
        zero_v[r, pl.ds(j * LANES, LANES)] = jnp.zeros((LANES,), jnp.float32)

    for z in range(rows_per_tile // zrows):
      pltpu.sync_copy(zero_v, acc_sh.at[pl.ds(r0 + z * zrows, zrows)])
    plsc.subcore_barrier()

    @pl.loop(0, n_chunks)
    def _(k):
      e0 = base + k * CHUNK
      pltpu.sync_copy(gidx_hbm.at[c, pl.ds(e0, CHUNK)], idx_v)
      pltpu.sync_copy(dst_hbm.at[pl.ds(e0, CHUNK)], dsti_v)
      pltpu.sync_copy(norm_hbm.at[pl.ds(e0, CHUNK)], norm_v)
      pltpu.sync_copy(v2_hbm.at[idx_v], rows_v)

      @pl.loop(0, CHUNK // LANES)
      def _(g):
        nv = norm_v[pl.ds(g * LANES, LANES)]
        for l in range(LANES):
          w = nv[l]
          r = g * LANES + l
          for j in range(dh // LANES):
            sl = pl.ds(j * LANES, LANES)
            rows_v[r, sl] = rows_v[r, sl] * w

      pltpu.sync_copy(rows_v, acc_sh.at[dsti_v], add=True)

    plsc.subcore_barrier()
    for z in range(rows_per_tile // zrows):
      rr = r0 + z * zrows
      pltpu.sync_copy(acc_sh.at[pl.ds(rr, zrows)],
                      out_hbm.at[pl.ds(rr, zrows), c])

  return lap


def _dis_body(part_ref, dis_ref):
  deg = part_ref[0, :, 0] + part_ref[1, :, 0]
  deg = deg[None, :]
  safe = jax.lax.rsqrt(jnp.maximum(deg, 1e-12))
  dis_ref[...] = jnp.where(deg > 0, safe, 0.0)


def _mm_init_body(x_ref, l_ref, w0_ref, w1_ref, o_ref):
  o_ref[...] = (
      jnp.dot(x_ref[...], w0_ref[...], preferred_element_type=jnp.float32)
      + jnp.dot(l_ref[...], w1_ref[...], preferred_element_type=jnp.float32))


def _mm_step_body(tprev_ref, l_ref, w_ref, acc_ref, t_ref, o_ref):
  t = 2.0 * l_ref[...] - tprev_ref[...]
  t_ref[...] = t
  o_ref[...] = acc_ref[...] + jnp.dot(
      t, w_ref[...], preferred_element_type=jnp.float32)


def _bn_body(acc_ref, b_ref, g_ref, bt_ref, o_ref):
  n = acc_ref.shape[0]
  t = acc_ref[...] + b_ref[...]
  mean = jnp.sum(t, axis=0, keepdims=True) / n
  d = t - mean
  var = jnp.sum(d * d, axis=0, keepdims=True) / n
  xn = d * jax.lax.rsqrt(var + EPS) * g_ref[...] + bt_ref[...]
  o_ref[...] = jnp.where(xn > 0, xn, ALPHA * xn)


def kernel(x, edge_idx, edge_attr, W, b, gamma, beta):
  n, din = x.shape
  kk, _, dout = W.shape
  e = edge_idx.shape[1]
  dh = din // NC

  e_pad = _cdiv(e, NW * CHUNK) * NW * CHUNK
  pad = e_pad - e
  src = jnp.concatenate([edge_idx[0], jnp.zeros((pad,), jnp.int32)])
  dst = jnp.concatenate([edge_idx[1], jnp.zeros((pad,), jnp.int32)])
  attr = jnp.concatenate([edge_attr, jnp.zeros((pad,), jnp.float32)])
  gidx = jnp.stack([NC * src, NC * src + 1])

  part = _deg_kernel(n, e_pad)(src, dst, attr)

  dis = pl.pallas_call(
      _dis_body,
      out_shape=jax.ShapeDtypeStruct((1, n), jnp.float32),
  )(part).reshape((n,))

  norm = _norm_kernel(n, e_pad)(src, dst, attr, dis)

  lap = _lap_kernel(n, e_pad, dh)

  def lap_apply(v):
    return lap(v.reshape((NC * n, dh)), gidx, dst, norm).reshape((n, din))

  blk = 2000
  grid = n // blk
  row_spec = pl.BlockSpec((blk, din), lambda i: (i, 0))
  w_spec = pl.BlockSpec((din, dout), lambda i: (0, 0))

  t1 = lap_apply(x)
  out = pl.pallas_call(
      _mm_init_body,
      grid=(grid,),
      in_specs=[row_spec, row_spec, w_spec, w_spec],
      out_specs=row_spec,
      out_shape=jax.ShapeDtypeStruct((n, dout), jnp.float32),
  )(x, t1, W[0], W[1])

  tprev, tcur = x, t1
  for k in range(2, kk):
    lk = lap_apply(tcur)
    tnext, out = pl.pallas_call(
        _mm_step_body,
        grid=(grid,),
        in_specs=[row_spec, row_spec, w_spec, row_spec],
        out_specs=[row_spec, row_spec],
        out_shape=[
            jax.ShapeDtypeStruct((n, din), jnp.float32),
            jax.ShapeDtypeStruct((n, dout), jnp.float32),
        ],
    )(tprev, lk, W[k], out)
    tprev, tcur = tcur, tnext

  vec_spec = pl.BlockSpec((1, dout), lambda: (0, 0))
  full_spec = pl.BlockSpec((n, dout), lambda: (0, 0))
  return pl.pallas_call(
      _bn_body,
      in_specs=[full_spec, vec_spec, vec_spec, vec_spec],
      out_specs=full_spec,
      out_shape=jax.ShapeDtypeStruct((n, dout), jnp.float32),
  )(out, b.reshape(1, dout), gamma.reshape(1, dout), beta.reshape(1, dout))


# speedup vs baseline: 4.2765x; 1.4064x over previous
import functools

import jax
import jax.numpy as jnp
from jax import lax
from jax.experimental import pallas as pl
from jax.experimental.pallas import tpu as pltpu
from jax.experimental.pallas import tpu_sc as plsc

ALPHA = 0.01
EPS = 1e-5

NC = 2
NS = 16
NW = NC * NS
LANES = 16
CHUNK = 128


def _cdiv(a, b):
  return (a + b - 1) // b


def _deg_kernel(n_nodes, e_pad):
  epw = e_pad // NW
  n_chunks = epw // CHUNK
  rows_per_tile = n_nodes // NS

  @functools.partial(
      pl.kernel,
      out_type=jax.ShapeDtypeStruct((NC, n_nodes, LANES), jnp.float32),
      mesh=plsc.VectorSubcoreMesh(core_axis_name="c", subcore_axis_name="s", num_cores=NC, num_subcores=NS),
      compiler_params=pltpu.CompilerParams(use_tc_tiling_on_sc=False, needs_layout_passes=False),
      scratch_types=[
          pltpu.MemorySpace.VMEM_SHARED((n_nodes, LANES), jnp.float32),
          pltpu.VMEM((rows_per_tile, LANES), jnp.float32),
          pltpu.VMEM((CHUNK, LANES), jnp.float32),
          pltpu.VMEM((CHUNK,), jnp.int32),
          pltpu.VMEM((CHUNK,), jnp.int32),
          pltpu.VMEM((CHUNK,), jnp.float32),
      ],
  )
  def deg(src_hbm, dst_hbm, attr_hbm, part_hbm,
          acc_sh, zero_v, rows_v, src_v, dst_v, attr_v):
    c = lax.axis_index("c")
    s = lax.axis_index("s")
    wid = s * NC + c
    base = wid * epw
    r0 = s * rows_per_tile

    @pl.loop(0, rows_per_tile)
    def _(r):
      zero_v[r, pl.ds(0, LANES)] = jnp.zeros((LANES,), jnp.float32)

    pltpu.sync_copy(zero_v, acc_sh.at[pl.ds(r0, rows_per_tile)])
    plsc.subcore_barrier()

    @pl.loop(0, n_chunks)
    def _(k):
      e0 = base + k * CHUNK
      pltpu.sync_copy(src_hbm.at[pl.ds(e0, CHUNK)], src_v)
      pltpu.sync_copy(dst_hbm.at[pl.ds(e0, CHUNK)], dst_v)
      pltpu.sync_copy(attr_hbm.at[pl.ds(e0, CHUNK)], attr_v)

      @pl.loop(0, CHUNK // LANES)
      def _(g):
        sl = pl.ds(g * LANES, LANES)
        ew = jnp.where(src_v[sl] == dst_v[sl], 0.0, attr_v[sl])
        for l in range(LANES):
          rows_v[g * LANES + l, pl.ds(0, LANES)] = jnp.full(
              (LANES,), ew[l], jnp.float32)

      pltpu.sync_copy(rows_v, acc_sh.at[src_v], add=True)

    plsc.subcore_barrier()
    pltpu.sync_copy(acc_sh.at[pl.ds(r0, rows_per_tile)],
                    part_hbm.at[c, pl.ds(r0, rows_per_tile)])

  return deg


def _norm_kernel(n_nodes, e_pad):
  epw = e_pad // NW
  n_chunks = epw // CHUNK

  @functools.partial(
      pl.kernel,
      out_type=jax.ShapeDtypeStruct((e_pad,), jnp.float32),
      mesh=plsc.VectorSubcoreMesh(core_axis_name="c", subcore_axis_name="s", num_cores=NC, num_subcores=NS),
      compiler_params=pltpu.CompilerParams(use_tc_tiling_on_sc=False, needs_layout_passes=False),
      scratch_types=[
          pltpu.VMEM((n_nodes,), jnp.float32),
          pltpu.VMEM((CHUNK,), jnp.int32),
          pltpu.VMEM((CHUNK,), jnp.int32),
          pltpu.VMEM((CHUNK,), jnp.float32),
          pltpu.VMEM((CHUNK,), jnp.float32),
      ],
  )
  def norm(src_hbm, dst_hbm, attr_hbm, dis_hbm, norm_hbm,
           dis_v, src_v, dst_v, attr_v, norm_v):
    c = lax.axis_index("c")
    s = lax.axis_index("s")
    wid = s * NC + c
    base = wid * epw
    pltpu.sync_copy(dis_hbm, dis_v)

    @pl.loop(0, n_chunks)
    def _(k):
      e0 = base + k * CHUNK
      pltpu.sync_copy(src_hbm.at[pl.ds(e0, CHUNK)], src_v)
      pltpu.sync_copy(dst_hbm.at[pl.ds(e0, CHUNK)], dst_v)
      pltpu.sync_copy(attr_hbm.at[pl.ds(e0, CHUNK)], attr_v)
      for j in range(CHUNK // LANES):
        sl = pl.ds(j * LANES, LANES)
        si = src_v[sl]
        di = dst_v[sl]
        av = attr_v[sl]
        dsrc = plsc.load_gather(dis_v, [si])
        ddst = plsc.load_gather(dis_v, [di])
        ew = jnp.where(si == di, 0.0, av)
        norm_v[sl] = -dsrc * ew * ddst
      pltpu.sync_copy(norm_v, norm_hbm.at[pl.ds(e0, CHUNK)])

  return norm


def _lap_kernel(n_nodes, e_pad, dh):
  epw = e_pad // NS
  n_chunks = epw // CHUNK
  rows_per_tile = n_nodes // NS
  zrows = 125

  @functools.partial(
      pl.kernel,
      out_type=jax.ShapeDtypeStruct((n_nodes, NC, dh), jnp.float32),
      mesh=plsc.VectorSubcoreMesh(core_axis_name="c", subcore_axis_name="s", num_cores=NC, num_subcores=NS),
      compiler_params=pltpu.CompilerParams(use_tc_tiling_on_sc=False, needs_layout_passes=False),
      scratch_types=[
          pltpu.MemorySpace.VMEM_SHARED((n_nodes, dh), jnp.float32),
          pltpu.VMEM((CHUNK, dh), jnp.float32),
          pltpu.VMEM((CHUNK, dh), jnp.float32),
          pltpu.VMEM((n_chunks, CHUNK), jnp.int32),
          pltpu.VMEM((2, CHUNK), jnp.int32),
          pltpu.VMEM((2, CHUNK), jnp.float32),
          pltpu.SemaphoreType.DMA,
          pltpu.SemaphoreType.DMA,
          pltpu.SemaphoreType.DMA,
          pltpu.SemaphoreType.DMA,
          pltpu.SemaphoreType.DMA,
          pltpu.SemaphoreType.DMA,
      ],
  )
  def lap(v2_hbm, gidx_hbm, dst_hbm, norm_hbm, out_hbm,
          acc_sh, rows0, rows1, idx_a, dst_b, norm_b,
          gsem0, gsem1, ssem0, ssem1, nsem0, nsem1):
    c = lax.axis_index("c")
    s = lax.axis_index("s")
    r0 = s * rows_per_tile

    pltpu.sync_copy(gidx_hbm.at[c, s], idx_a)

    @pl.loop(0, zrows)
    def _(r):
      for j in range(dh // LANES):
        rows0[r, pl.ds(j * LANES, LANES)] = jnp.zeros((LANES,), jnp.float32)

    for z in range(rows_per_tile // zrows):
      pltpu.sync_copy(rows0.at[pl.ds(0, zrows)],
                      acc_sh.at[pl.ds(r0 + z * zrows, zrows)])
    plsc.subcore_barrier()

    def gather(k, rows, sem):
      pltpu.async_copy(v2_hbm.at[idx_a.at[k]], rows, sem)

    def gwait(k, rows, sem):
      pltpu.make_async_copy(v2_hbm.at[idx_a.at[k]], rows, sem).wait()

    def ndfetch(k, b, sem):
      pltpu.async_copy(dst_hbm.at[s, k], dst_b.at[b], sem)
      pltpu.async_copy(norm_hbm.at[s, k], norm_b.at[b], sem)

    def ndwait(k, b, sem):
      pltpu.make_async_copy(dst_hbm.at[s, k], dst_b.at[b], sem).wait()
      pltpu.make_async_copy(norm_hbm.at[s, k], norm_b.at[b], sem).wait()

    def scat(k, b, rows, sem):
      pltpu.async_copy(rows, acc_sh.at[dst_b.at[b]], sem, add=True)

    def swait(k, b, rows, sem):
      pltpu.make_async_copy(rows, acc_sh.at[dst_b.at[b]], sem).wait()

    def mul(b, rows):
      @pl.loop(0, CHUNK // LANES)
      def _(g):
        nv = norm_b[b, pl.ds(g * LANES, LANES)]
        for l in range(LANES):
          w = nv[l]
          r = g * LANES + l
          for j in range(dh // LANES):
            sl = pl.ds(j * LANES, LANES)
            rows[r, sl] = rows[r, sl] * w

    ndfetch(0, 0, nsem0)
    ndfetch(1, 1, nsem1)
    gather(0, rows0, gsem0)
    gather(1, rows1, gsem1)

    @pl.loop(0, n_chunks - 2, step=2)
    def _(k0):
      gwait(k0, rows0, gsem0)
      ndwait(k0, 0, nsem0)
      mul(0, rows0)
      scat(k0, 0, rows0, ssem0)
      gwait(k0 + 1, rows1, gsem1)
      ndwait(k0 + 1, 1, nsem1)
      mul(1, rows1)
      scat(k0 + 1, 1, rows1, ssem1)
      swait(k0, 0, rows0, ssem0)
      ndfetch(k0 + 2, 0, nsem0)
      gather(k0 + 2, rows0, gsem0)
      swait(k0 + 1, 1, rows1, ssem1)
      ndfetch(k0 + 3, 1, nsem1)
      gather(k0 + 3, rows1, gsem1)

    kl = n_chunks - 2
    gwait(kl, rows0, gsem0)
    ndwait(kl, 0, nsem0)
    mul(0, rows0)
    scat(kl, 0, rows0, ssem0)
    gwait(kl + 1, rows1, gsem1)
    ndwait(kl + 1, 1, nsem1)
    mul(1, rows1)
    scat(kl + 1, 1, rows1, ssem1)
    swait(kl, 0, rows0, ssem0)
    swait(kl + 1, 1, rows1, ssem1)

    plsc.subcore_barrier()
    for z in range(rows_per_tile // zrows):
      rr = r0 + z * zrows
      pltpu.sync_copy(acc_sh.at[pl.ds(rr, zrows)],
                      out_hbm.at[pl.ds(rr, zrows), c])

  return lap


def _dis_body(part_ref, dis_ref):
  deg = part_ref[0, :, 0] + part_ref[1, :, 0]
  deg = deg[None, :]
  safe = jax.lax.rsqrt(jnp.maximum(deg, 1e-12))
  dis_ref[...] = jnp.where(deg > 0, safe, 0.0)


def _mm_init_body(x_ref, l_ref, w0_ref, w1_ref, o_ref):
  o_ref[...] = (
      jnp.dot(x_ref[...], w0_ref[...], preferred_element_type=jnp.float32)
      + jnp.dot(l_ref[...], w1_ref[...], preferred_element_type=jnp.float32))


def _mm_step_body(tprev_ref, l_ref, w_ref, acc_ref, t_ref, o_ref):
  t = 2.0 * l_ref[...] - tprev_ref[...]
  t_ref[...] = t
  o_ref[...] = acc_ref[...] + jnp.dot(
      t, w_ref[...], preferred_element_type=jnp.float32)


def _bn_body(acc_ref, b_ref, g_ref, bt_ref, o_ref):
  n = acc_ref.shape[0]
  t = acc_ref[...] + b_ref[...]
  mean = jnp.sum(t, axis=0, keepdims=True) / n
  d = t - mean
  var = jnp.sum(d * d, axis=0, keepdims=True) / n
  xn = d * jax.lax.rsqrt(var + EPS) * g_ref[...] + bt_ref[...]
  o_ref[...] = jnp.where(xn > 0, xn, ALPHA * xn)


def kernel(x, edge_idx, edge_attr, W, b, gamma, beta):
  n, din = x.shape
  kk, _, dout = W.shape
  e = edge_idx.shape[1]
  dh = din // NC

  e_pad = _cdiv(e, NW * CHUNK) * NW * CHUNK
  pad = e_pad - e
  src = jnp.concatenate([edge_idx[0], jnp.zeros((pad,), jnp.int32)])
  dst = jnp.concatenate([edge_idx[1], jnp.zeros((pad,), jnp.int32)])
  attr = jnp.concatenate([edge_attr, jnp.zeros((pad,), jnp.float32)])
  gidx = jnp.stack([NC * src, NC * src + 1])

  part = _deg_kernel(n, e_pad)(src, dst, attr)

  dis = pl.pallas_call(
      _dis_body,
      out_shape=jax.ShapeDtypeStruct((1, n), jnp.float32),
  )(part).reshape((n,))

  norm = _norm_kernel(n, e_pad)(src, dst, attr, dis)

  lap = _lap_kernel(n, e_pad, dh)
  n_chunks = e_pad // NS // CHUNK
  gidx_l = gidx.reshape(NC, NS, n_chunks, CHUNK)
  dst_l = dst.reshape(NS, n_chunks, CHUNK)
  norm_l = norm.reshape(NS, n_chunks, CHUNK)

  def lap_apply(v):
    return lap(v.reshape((NC * n, dh)), gidx_l, dst_l, norm_l).reshape((n, din))

  blk = 2000
  grid = n // blk
  row_spec = pl.BlockSpec((blk, din), lambda i: (i, 0))
  w_spec = pl.BlockSpec((din, dout), lambda i: (0, 0))

  t1 = lap_apply(x)
  out = pl.pallas_call(
      _mm_init_body,
      grid=(grid,),
      in_specs=[row_spec, row_spec, w_spec, w_spec],
      out_specs=row_spec,
      out_shape=jax.ShapeDtypeStruct((n, dout), jnp.float32),
  )(x, t1, W[0], W[1])

  tprev, tcur = x, t1
  for k in range(2, kk):
    lk = lap_apply(tcur)
    tnext, out = pl.pallas_call(
        _mm_step_body,
        grid=(grid,),
        in_specs=[row_spec, row_spec, w_spec, row_spec],
        out_specs=[row_spec, row_spec],
        out_shape=[
            jax.ShapeDtypeStruct((n, din), jnp.float32),
            jax.ShapeDtypeStruct((n, dout), jnp.float32),
        ],
    )(tprev, lk, W[k], out)
    tprev, tcur = tcur, tnext

  vec_spec = pl.BlockSpec((1, dout), lambda: (0, 0))
  full_spec = pl.BlockSpec((n, dout), lambda: (0, 0))
  return pl.pallas_call(
      _bn_body,
      in_specs=[full_spec, vec_spec, vec_spec, vec_spec],
      out_specs=full_spec,
      out_shape=jax.ShapeDtypeStruct((n, dout), jnp.float32),
  )(out, b.reshape(1, dout), gamma.reshape(1, dout), beta.reshape(1, dout))
